# Initial kernel scaffold; baseline (speedup 1.0000x reference)
import functools

import jax
import jax.numpy as jnp
from jax import lax
from jax.experimental import pallas as pl
from jax.experimental.pallas import tpu as pltpu
from jax.experimental.pallas import tpu_sc as plsc

_K = 8
_N = 64
_BSZ, _WINDOW, _ENSEMBLE = 8, 2048, 64
_ROWS = _BSZ * _WINDOW
_L = 16
_NC, _NS = 2, 16
_NW = _NC * _NS
_RPW = _ROWS // _NW
_CHUNK = 128
_NCHUNK = _RPW // _CHUNK
_NGRP = _CHUNK // _L
_NEG = jnp.float32(-1e30)


def _gumbel_noise():
    u = jax.random.uniform(jax.random.key(42), (1, _ROWS, _N),
                           minval=1e-9, maxval=1.0, dtype=jnp.float32)
    return (-jnp.log(-jnp.log(u)))[0]


_GUMBEL = _gumbel_noise()


def _sc_body(x_hbm, g_hbm, mask_hbm, marg_hbm,
             xv, gv, maskv, margv, xT, pertT, wT, Fv):
    cid = lax.axis_index("c")
    sid = lax.axis_index("s")
    wid = sid * _NC + cid
    base = wid * _RPW

    def full16(s):
        return jnp.full((_L,), s, dtype=jnp.int32)

    def do_group(grp, carry_g):
        ridx = lax.iota(jnp.int32, _L) + grp * _L

        def pre_body(i, carry):
            mx, t = carry
            ii = full16(i)
            xi = plsc.load_gather(xv, [ridx, ii])
            gi = plsc.load_gather(gv, [ridx, ii])
            xT[i, :] = xi
            p = xi + gi
            pertT[i, :] = p
            mx = jnp.maximum(mx, xi)
            t = list(t)
            for j in range(_K):
                hi = jnp.maximum(t[j], p)
                p = jnp.minimum(t[j], p)
                t[j] = hi
            return mx, tuple(t)

        neg = jnp.full((_L,), _NEG, dtype=jnp.float32)
        mx, tops = lax.fori_loop(0, _N, pre_body, (neg, (neg,) * _K))
        thr = tops[_K - 1]

        def fwd_body(i, E):
            xi = xT[i, :]
            wi = jnp.exp(xi - mx)
            wT[i, :] = wi
            for j in range(1, _K):
                Fv[i, j - 1, :] = E[j - 1]
            newE = list(E)
            for j in range(_K - 1, 0, -1):
                newE[j] = E[j] + E[j - 1] * wi
            newE[0] = E[0] + wi
            return tuple(newE)

        zero = jnp.zeros((_L,), dtype=jnp.float32)
        E = lax.fori_loop(0, _N, fwd_body, (zero,) * _K)
        invZ = 1.0 / E[_K - 1]

        def bwd_body(t, C):
            i = _N - 1 - t
            wi = wT[i, :]
            e = C[0]
            for j in range(1, _K):
                e = e + Fv[i, j - 1, :] * C[j]
            p = wi * e * invZ
            ii = full16(i)
            plsc.store_scatter(margv, [ridx, ii], p)
            pe = pertT[i, :]
            mk = jnp.where(pe >= thr, jnp.float32(1.0), jnp.float32(0.0))
            plsc.store_scatter(maskv, [ridx, ii], mk)
            newC = list(C)
            for m in range(_K - 2, -1, -1):
                newC[m] = C[m] + C[m + 1] * wi
            return tuple(newC)

        one = jnp.ones((_L,), dtype=jnp.float32)
        lax.fori_loop(0, _N, bwd_body, (zero,) * (_K - 1) + (one,))
        return carry_g

    def do_chunk(c, carry_c):
        r0 = base + c * _CHUNK
        pltpu.sync_copy(x_hbm.at[pl.ds(r0, _CHUNK), :], xv)
        pltpu.sync_copy(g_hbm.at[pl.ds(r0, _CHUNK), :], gv)
        lax.fori_loop(0, _NGRP, do_group, 0)
        pltpu.sync_copy(maskv, mask_hbm.at[pl.ds(r0, _CHUNK), :])
        pltpu.sync_copy(margv, marg_hbm.at[pl.ds(r0, _CHUNK), :])
        return carry_c

    lax.fori_loop(0, _NCHUNK, do_chunk, 0)


@functools.partial(
    pl.kernel,
    out_type=(jax.ShapeDtypeStruct((_ROWS, _N), jnp.float32),
              jax.ShapeDtypeStruct((_ROWS, _N), jnp.float32)),
    mesh=plsc.VectorSubcoreMesh(core_axis_name="c", subcore_axis_name="s",
                                num_cores=_NC, num_subcores=_NS),
    scratch_types=[
        pltpu.VMEM((_CHUNK, _N), jnp.float32),
        pltpu.VMEM((_CHUNK, _N), jnp.float32),
        pltpu.VMEM((_CHUNK, _N), jnp.float32),
        pltpu.VMEM((_CHUNK, _N), jnp.float32),
        pltpu.VMEM((_N, _L), jnp.float32),
        pltpu.VMEM((_N, _L), jnp.float32),
        pltpu.VMEM((_N, _L), jnp.float32),
        pltpu.VMEM((_N, _K - 1, _L), jnp.float32),
    ],
)
def _sc_kernel(x_hbm, g_hbm, mask_hbm, marg_hbm, *scratch):
    _sc_body(x_hbm, g_hbm, mask_hbm, marg_hbm, *scratch)


def kernel(scores):
    flat = scores.reshape(_ROWS, _N)
    mask, marg = _sc_kernel(flat, _GUMBEL)
    return (mask.reshape(_BSZ, _WINDOW, _ENSEMBLE),
            marg.reshape(_BSZ, _WINDOW, _ENSEMBLE))


# speedup vs baseline: 5.2273x; 5.2273x over previous
import functools

import jax
import jax.numpy as jnp
import numpy as np
from jax import lax
from jax.experimental import pallas as pl
from jax.experimental.pallas import tpu as pltpu
from jax.experimental.pallas import tpu_sc as plsc

_K = 8
_N = 64
_BSZ, _WINDOW, _ENSEMBLE = 8, 2048, 64
_ROWS = _BSZ * _WINDOW
_L = 16
_NC, _NS = 2, 16
_NW = _NC * _NS
_RPW = _ROWS // _NW
_CHUNK = 128
_NCHUNK = _RPW // _CHUNK
_NGRP = _CHUNK // _L
_CELEM = _CHUNK * _N
_NEG = -1e30


def _rotl32(x, d):
    return ((x << np.uint32(d)) | (x >> np.uint32(32 - d))).astype(np.uint32)


def _threefry2x32_np(k0, k1, x0, x1):
    rot = ((13, 15, 26, 6), (17, 29, 16, 24))
    ks = (np.uint32(k0), np.uint32(k1),
          np.uint32(np.uint32(k0) ^ np.uint32(k1) ^ np.uint32(0x1BD11BDA)))
    x0 = (x0 + ks[0]).astype(np.uint32)
    x1 = (x1 + ks[1]).astype(np.uint32)
    for r in range(5):
        for d in rot[r % 2]:
            x0 = (x0 + x1).astype(np.uint32)
            x1 = x0 ^ _rotl32(x1, d)
        x0 = (x0 + ks[(r + 1) % 3]).astype(np.uint32)
        x1 = (x1 + ks[(r + 2) % 3] + np.uint32(r + 1)).astype(np.uint32)
    return x0, x1


def _uniform_noise_np():
    size = _ROWS * _N
    idx = np.arange(size, dtype=np.uint64)
    c_hi = (idx >> np.uint64(32)).astype(np.uint32)
    c_lo = (idx & np.uint64(0xFFFFFFFF)).astype(np.uint32)
    x0, x1 = _threefry2x32_np(np.uint32(0), np.uint32(42), c_hi, c_lo)
    bits = x0 ^ x1
    float_bits = (bits >> np.uint32(9)) | np.uint32(0x3F800000)
    floats = float_bits.view(np.float32) - np.float32(1.0)
    mn, mx = np.float32(1e-9), np.float32(1.0)
    return np.maximum(mn, (floats * (mx - mn) + mn).astype(np.float32))


_UNIFORM = _uniform_noise_np()


def _sc_body(x_hbm, g_hbm, mask_hbm, marg_hbm,
             xv, gv, maskv, margv, xT, pertT, wT, Fv):
    cid = lax.axis_index("c")
    sid = lax.axis_index("s")
    wid = sid * _NC + cid
    base = wid * _RPW

    def do_group(grp, carry_g):
        ridx64 = (lax.iota(jnp.int32, _L) + grp * _L) * _N

        def pre_body(i, carry):
            mx, t = carry
            ii = ridx64 + i
            xi = plsc.load_gather(xv, [ii])
            gi = plsc.load_gather(gv, [ii])
            xT[pl.ds(i * _L, _L)] = xi
            p = xi + gi
            pertT[pl.ds(i * _L, _L)] = p
            mx = jnp.maximum(mx, xi)
            t = list(t)
            for j in range(_K):
                hi = jnp.maximum(t[j], p)
                p = jnp.minimum(t[j], p)
                t[j] = hi
            return mx, tuple(t)

        neg = jnp.full((_L,), jnp.float32(_NEG), dtype=jnp.float32)
        mx, tops = lax.fori_loop(0, _N, pre_body, (neg, (neg,) * _K))
        thr = tops[_K - 1]

        def fwd_body(i, E):
            xi = xT[pl.ds(i * _L, _L)]
            wi = jnp.exp(xi - mx)
            wT[pl.ds(i * _L, _L)] = wi
            for j in range(1, _K):
                Fv[pl.ds((i * (_K - 1) + j - 1) * _L, _L)] = E[j - 1]
            newE = list(E)
            for j in range(_K - 1, 0, -1):
                newE[j] = E[j] + E[j - 1] * wi
            newE[0] = E[0] + wi
            return tuple(newE)

        zero = jnp.zeros((_L,), dtype=jnp.float32)
        E = lax.fori_loop(0, _N, fwd_body, (zero,) * _K)
        invZ = 1.0 / E[_K - 1]

        def bwd_body(t, C):
            i = _N - 1 - t
            wi = wT[pl.ds(i * _L, _L)]
            e = C[0]
            for j in range(1, _K):
                e = e + Fv[pl.ds((i * (_K - 1) + j - 1) * _L, _L)] * C[j]
            p = wi * e * invZ
            ii = ridx64 + i
            plsc.store_scatter(margv, [ii], p)
            pe = pertT[pl.ds(i * _L, _L)]
            mk = jnp.where(pe >= thr, jnp.float32(1.0), jnp.float32(0.0))
            plsc.store_scatter(maskv, [ii], mk)
            newC = list(C)
            for m in range(_K - 2, -1, -1):
                newC[m] = C[m] + C[m + 1] * wi
            return tuple(newC)

        one = jnp.ones((_L,), dtype=jnp.float32)
        lax.fori_loop(0, _N, bwd_body, (zero,) * (_K - 1) + (one,))
        return carry_g

    def do_chunk(c, carry_c):
        e0 = (base + c * _CHUNK) * _N
        pltpu.sync_copy(x_hbm.at[pl.ds(e0, _CELEM)], xv)
        pltpu.sync_copy(g_hbm.at[pl.ds(e0, _CELEM)], gv)
        lax.fori_loop(0, _NGRP, do_group, 0)
        pltpu.sync_copy(maskv, mask_hbm.at[pl.ds(e0, _CELEM)])
        pltpu.sync_copy(margv, marg_hbm.at[pl.ds(e0, _CELEM)])
        return carry_c

    lax.fori_loop(0, _NCHUNK, do_chunk, 0)


@functools.partial(
    pl.kernel,
    out_type=(jax.ShapeDtypeStruct((_ROWS * _N,), jnp.float32),
              jax.ShapeDtypeStruct((_ROWS * _N,), jnp.float32)),
    mesh=plsc.VectorSubcoreMesh(core_axis_name="c", subcore_axis_name="s",
                                num_cores=_NC, num_subcores=_NS),
    scratch_types=[
        pltpu.VMEM((_CELEM,), jnp.float32),
        pltpu.VMEM((_CELEM,), jnp.float32),
        pltpu.VMEM((_CELEM,), jnp.float32),
        pltpu.VMEM((_CELEM,), jnp.float32),
        pltpu.VMEM((_N * _L,), jnp.float32),
        pltpu.VMEM((_N * _L,), jnp.float32),
        pltpu.VMEM((_N * _L,), jnp.float32),
        pltpu.VMEM((_N * (_K - 1) * _L,), jnp.float32),
    ],
    compiler_params=pltpu.CompilerParams(needs_layout_passes=False),
)
def _sc_kernel(x_hbm, g_hbm, mask_hbm, marg_hbm, *scratch):
    _sc_body(x_hbm, g_hbm, mask_hbm, marg_hbm, *scratch)


def kernel(scores):
    flat = scores.reshape(_ROWS * _N)
    u = lax.optimization_barrier(jnp.asarray(_UNIFORM))
    gumbel = -jnp.log(-jnp.log(u))
    mask, marg = _sc_kernel(flat, gumbel)
    return (mask.reshape(_BSZ, _WINDOW, _ENSEMBLE),
            marg.reshape(_BSZ, _WINDOW, _ENSEMBLE))
